# Initial kernel scaffold; baseline (speedup 1.0000x reference)
import jax
import jax.numpy as jnp
from jax.experimental import pallas as pl


def kernel(x, edge_index, rest_lengths):
    raise NotImplementedError("write your pallas kernel here")


# speedup vs baseline: 581.2813x; 581.2813x over previous
import functools
import math

import jax
import jax.numpy as jnp
from jax import lax
from jax.experimental import pallas as pl
from jax.experimental.pallas import tpu as pltpu
from jax.experimental.pallas import tpu_sc as plsc

_ITERS = 6
_MAX_CORR = 0.2
_NC = 2
_NS = 16
_ROW = 128
_R = 8
_K = _R * _ROW


def _rsqrt(s):
    i = lax.bitcast_convert_type(s, jnp.int32)
    i = jnp.int32(0x5F3759DF) - lax.shift_right_logical(i, 1)
    y = lax.bitcast_convert_type(i, jnp.float32)
    for _ in range(3):
        y = y * (1.5 - 0.5 * s * y * y)
    return y


def _clip(v):
    return jnp.minimum(jnp.maximum(v, -_MAX_CORR), _MAX_CORR)


def _make_sc_call(B, N_pad, C):
    SL = N_pad // _NS

    mesh = plsc.VectorSubcoreMesh(
        core_axis_name="c", subcore_axis_name="s",
        num_cores=_NC, num_subcores=_NS)

    scratch = (
        [pltpu.VMEM_SHARED((N_pad,), jnp.float32) for _ in range(6)]
        + [pltpu.VMEM((_R, _ROW), jnp.int32) for _ in range(2)]
        + [pltpu.VMEM((_K,), jnp.float32) for _ in range(13)]
        + [pltpu.VMEM((SL,), jnp.float32),
           pltpu.SemaphoreType.DMA, pltpu.SemaphoreType.DMA]
    )

    @functools.partial(
        pl.kernel,
        out_type=jax.ShapeDtypeStruct((B * 3 * N_pad,), jnp.float32),
        mesh=mesh,
        scratch_types=scratch,
    )
    def run(xT_hbm, ei_hbm, rl_hbm, out_hbm,
            ax, ay, az, bx, by, bz,
            sidx, didx,
            l0b, gsx, gsy, gsz, gdx, gdy, gdz,
            cpx, cpy, cpz, cnx, cny, cnz,
            ub, gsem, ssem):
        c = lax.axis_index("c")
        s = lax.axis_index("s")
        lo = s * SL
        a_planes = (ax, ay, az)
        b_planes = (bx, by, bz)

        for comp in range(3):
            hb = pl.ds((c * 3 + comp) * N_pad + lo, SL)
            pltpu.sync_copy(xT_hbm.at[hb], ub)
            pltpu.sync_copy(ub, a_planes[comp].at[pl.ds(lo, SL)])
            pltpu.sync_copy(ub, b_planes[comp].at[pl.ds(lo, SL)])
        plsc.subcore_barrier()

        def compute_body(v, _):
            d = pl.ds(v * 16, 16)
            xs, ys, zs = gsx[d], gsy[d], gsz[d]
            xd, yd, zd = gdx[d], gdy[d], gdz[d]
            l0 = l0b[d]
            dx = xs - xd
            dy = ys - yd
            dz = zs - zd
            s2 = jnp.maximum(dx * dx + dy * dy + dz * dz, 1e-30)
            y = _rsqrt(s2)
            dist = s2 * y
            t = (dist - l0) * y * (-0.5)
            cx = _clip(dx * t)
            cy = _clip(dy * t)
            cz = _clip(dz * t)
            cpx[d] = cx
            cpy[d] = cy
            cpz[d] = cz
            cnx[d] = -cx
            cny[d] = -cy
            cnz[d] = -cz
            return _

        def chunk_body(ch, _):
            pltpu.sync_copy(ei_hbm.at[0, s, ch], sidx)
            pltpu.sync_copy(ei_hbm.at[1, s, ch], didx)
            pltpu.sync_copy(rl_hbm.at[pl.ds((s * C + ch) * _K, _K)], l0b)
            descs = []
            for r in range(_R):
                d = pl.ds(r * _ROW, _ROW)
                for plane, g in ((ax, gsx), (ay, gsy), (az, gsz)):
                    descs.append(
                        pltpu.async_copy(plane.at[sidx.at[r]], g.at[d], gsem))
                for plane, g in ((ax, gdx), (ay, gdy), (az, gdz)):
                    descs.append(
                        pltpu.async_copy(plane.at[didx.at[r]], g.at[d], gsem))
            for desc in descs:
                desc.wait()
            lax.fori_loop(0, _K // 16, compute_body, None)
            descs = []
            for r in range(_R):
                d = pl.ds(r * _ROW, _ROW)
                for plane, cb in ((bx, cpx), (by, cpy), (bz, cpz)):
                    descs.append(pltpu.async_copy(
                        cb.at[d], plane.at[sidx.at[r]], ssem, add=True))
                for plane, cb in ((bx, cnx), (by, cny), (bz, cnz)):
                    descs.append(pltpu.async_copy(
                        cb.at[d], plane.at[didx.at[r]], ssem, add=True))
            for desc in descs:
                desc.wait()
            return _

        for it in range(_ITERS):
            lax.fori_loop(0, C, chunk_body, None)
            plsc.subcore_barrier()
            if it < _ITERS - 1:
                for comp in range(3):
                    pltpu.sync_copy(b_planes[comp].at[pl.ds(lo, SL)], ub)
                    pltpu.sync_copy(ub, a_planes[comp].at[pl.ds(lo, SL)])
                plsc.subcore_barrier()

        for comp in range(3):
            pltpu.sync_copy(b_planes[comp].at[pl.ds(lo, SL)], ub)
            pltpu.sync_copy(ub, out_hbm.at[pl.ds((c * 3 + comp) * N_pad + lo, SL)])

    return run


def kernel(x, edge_index, rest_lengths):
    B, N, _ = x.shape
    E = edge_index.shape[1]
    N_pad = math.ceil(N / (_NS * 8)) * (_NS * 8)
    C = math.ceil(E / (_NS * _K))
    E_pad = C * _NS * _K

    xT = jnp.pad(jnp.transpose(x, (0, 2, 1)), ((0, 0), (0, 0), (0, N_pad - N)))
    ei = jnp.pad(edge_index.astype(jnp.int32), ((0, 0), (0, E_pad - E)))
    ei = ei.reshape(2, _NS, C, _R, _ROW)
    rl = jnp.pad(rest_lengths, (0, E_pad - E), constant_values=1.0)

    out = _make_sc_call(B, N_pad, C)(xT.reshape(-1), ei, rl)
    out = out.reshape(B, 3, N_pad)
    return jnp.transpose(out[:, :, :N], (0, 2, 1))
